# baseline (speedup 1.0000x reference)
import jax
import jax.numpy as jnp
from jax import lax
from jax.experimental import pallas as pl
from jax.experimental.pallas import tpu as pltpu
from jax.experimental.pallas import tpu_sc as plsc

VOCAB = 100000
WEAPON_VOCAB = 1000
B = 1024
HIST = 200
EMB = 4

NUM_CORES = 2
NUM_SUBCORES = 16
NW = NUM_CORES * NUM_SUBCORES
B_PER_W = B // NW
E_PER_W = B_PER_W * HIST * EMB
O_PER_W = B_PER_W * EMB
NVEC = 16


def _sc_pool_body(ab_table, ab_idx, wp_table, wp_idx, x_out,
                  idx2_v, idx4_v, rows_v, widx_v, widx4_v, wrows_v, out_v,
                  sem):
    wid = lax.axis_index("s") * NUM_CORES + lax.axis_index("c")

    pltpu.sync_copy(ab_idx.at[pl.ds(wid * B_PER_W, B_PER_W)], idx2_v)
    pltpu.sync_copy(wp_idx.at[pl.ds(wid * B_PER_W, B_PER_W)], widx_v)

    lanes = lax.iota(jnp.int32, NVEC)
    sub = lanes >> 2
    elem = lanes & 3
    zero = jnp.zeros((NVEC,), jnp.int32)

    SEC = B_PER_W * HIST

    def exp_body(t, carry):
        rowv, colv = carry
        m = plsc.load_gather(idx2_v, [rowv, colv]) * EMB
        base = t * NVEC
        idx4_v[pl.ds(base, NVEC)] = m
        idx4_v[pl.ds(SEC + base, NVEC)] = m + 1
        idx4_v[pl.ds(2 * SEC + base, NVEC)] = m + 2
        idx4_v[pl.ds(3 * SEC + base, NVEC)] = m + 3
        colv = colv + NVEC
        wrap = colv >= HIST
        colv = jnp.where(wrap, colv - HIST, colv)
        return rowv + wrap.astype(jnp.int32), colv

    lax.fori_loop(0, SEC // NVEC, exp_body, (zero, lanes))

    copy = pltpu.async_copy(ab_table.at[idx4_v], rows_v, sem)

    for v in range(O_PER_W // NVEC):
        wvals = plsc.load_gather(widx_v, [v * 4 + sub, zero])
        widx4_v[pl.ds(v * NVEC, NVEC)] = wvals * EMB + elem
    wcopy = pltpu.async_copy(wp_table.at[widx4_v], wrows_v, sem)
    copy.wait()
    wcopy.wait()

    for v in range(O_PER_W // NVEC):
        acc = wrows_v[pl.ds(v * NVEC, NVEC)]
        ebase = elem * SEC + (v * 4 + sub) * HIST

        def body(i, acc):
            i4 = i * 4
            g0 = plsc.load_gather(rows_v, [ebase + i4])
            g1 = plsc.load_gather(rows_v, [ebase + (i4 + 1)])
            g2 = plsc.load_gather(rows_v, [ebase + (i4 + 2)])
            g3 = plsc.load_gather(rows_v, [ebase + (i4 + 3)])
            return acc + ((g0 + g1) + (g2 + g3))

        acc = lax.fori_loop(0, HIST // 4, body, acc)
        out_v[pl.ds(v * NVEC, NVEC)] = acc

    pltpu.sync_copy(out_v, x_out.at[pl.ds(wid * O_PER_W, O_PER_W)])


def _sc_pool(ab_idx, wp_idx, at_flat, wt_flat):
    mesh = plsc.VectorSubcoreMesh(core_axis_name="c", subcore_axis_name="s",
                                  num_cores=NUM_CORES,
                                  num_subcores=NUM_SUBCORES)
    fn = pl.kernel(
        _sc_pool_body,
        out_type=jax.ShapeDtypeStruct((B * EMB,), jnp.float32),
        mesh=mesh,
        compiler_params=pltpu.CompilerParams(needs_layout_passes=False),
        scratch_types=[
            pltpu.VMEM((B_PER_W, HIST), jnp.int32),
            pltpu.VMEM((E_PER_W,), jnp.int32),
            pltpu.VMEM((E_PER_W,), jnp.float32),
            pltpu.VMEM((B_PER_W, 1), jnp.int32),
            pltpu.VMEM((O_PER_W,), jnp.int32),
            pltpu.VMEM((O_PER_W,), jnp.float32),
            pltpu.VMEM((O_PER_W,), jnp.float32),
            pltpu.SemaphoreType.DMA,
        ],
    )
    return fn(at_flat, ab_idx, wt_flat, wp_idx)


B_TILE = 16
NBUF = 4


def _tc_linear_body(x_ref, w_ref, b_ref, o_hbm, obuf, sems):
    i = pl.program_id(0)
    nb = pl.num_programs(0)
    s = lax.rem(i, NBUF)

    @pl.when(i >= NBUF)
    def _():
        pltpu.make_async_copy(
            obuf.at[s], o_hbm.at[pl.ds((i - NBUF) * B_TILE, B_TILE)],
            sems.at[s]).wait()

    obuf[s] = lax.dot_general(
        x_ref[...], w_ref[...], (((1,), (0,)), ((), ())),
        preferred_element_type=jnp.float32) + b_ref[...]
    pltpu.async_copy(obuf.at[s], o_hbm.at[pl.ds(i * B_TILE, B_TILE)],
                     sems.at[s])

    @pl.when(i == nb - 1)
    def _():
        for t in range(NBUF):
            j = nb - NBUF + t
            pltpu.make_async_copy(
                obuf.at[lax.rem(j, NBUF)],
                o_hbm.at[pl.ds(j * B_TILE, B_TILE)],
                sems.at[lax.rem(j, NBUF)]).wait()


def _tc_linear(x2d, W, b2d):
    nb = B // B_TILE
    return pl.pallas_call(
        _tc_linear_body,
        grid=(nb,),
        in_specs=[
            pl.BlockSpec((B_TILE, EMB), lambda i: (i, 0)),
            pl.BlockSpec((EMB, VOCAB), lambda i: (0, 0)),
            pl.BlockSpec((1, VOCAB), lambda i: (0, 0)),
        ],
        out_specs=pl.BlockSpec(memory_space=pl.ANY),
        out_shape=jax.ShapeDtypeStruct((B, VOCAB), jnp.float32),
        scratch_shapes=[
            pltpu.VMEM((NBUF, B_TILE, VOCAB), jnp.float32),
            pltpu.SemaphoreType.DMA((NBUF,)),
        ],
    )(x2d, W, b2d)


def kernel(abilities, weapons, ability_table, weapon_table, W, b):
    ab_idx = abilities if abilities.dtype == jnp.int32 else (
        abilities.astype(jnp.int32))
    wp_idx = weapons if weapons.dtype == jnp.int32 else (
        weapons.astype(jnp.int32))
    x = _sc_pool(ab_idx, wp_idx,
                 ability_table.reshape(-1), weapon_table.reshape(-1))
    x2d = x.reshape(B, EMB)
    return _tc_linear(x2d, W, b.reshape(1, VOCAB))


# speedup vs baseline: 1.0040x; 1.0040x over previous
import jax
import jax.numpy as jnp
from jax import lax
from jax.experimental import pallas as pl
from jax.experimental.pallas import tpu as pltpu
from jax.experimental.pallas import tpu_sc as plsc

VOCAB = 100000
WEAPON_VOCAB = 1000
B = 1024
HIST = 200
EMB = 4

NUM_CORES = 2
NUM_SUBCORES = 16
NW = NUM_CORES * NUM_SUBCORES
B_PER_W = B // NW
E_PER_W = B_PER_W * HIST * EMB
O_PER_W = B_PER_W * EMB
NVEC = 16


def _sc_pool_body(ab_table, ab_idx, wp_table, wp_idx, x_out,
                  idx2_v, i40, i41, i42, i43, r0, r1, r2, r3,
                  widx_v, widx4_v, wrows_v, out_v,
                  s0, s1, s2, s3, wsem):
    wid = lax.axis_index("s") * NUM_CORES + lax.axis_index("c")

    pltpu.sync_copy(ab_idx.at[pl.ds(wid * B_PER_W, B_PER_W)], idx2_v)
    pltpu.sync_copy(wp_idx.at[pl.ds(wid * B_PER_W, B_PER_W)], widx_v)

    lanes = lax.iota(jnp.int32, NVEC)
    sub = lanes >> 2
    elem = lanes & 3
    zero = jnp.zeros((NVEC,), jnp.int32)

    GROUPS = 4
    GB = B_PER_W // GROUPS
    GSEC = GB * HIST
    idx4_g = [i40, i41, i42, i43]
    rows_g = [r0, r1, r2, r3]
    sem_g = [s0, s1, s2, s3]
    copies = []

    for g in range(GROUPS):
        i4v = idx4_g[g]

        def exp_body(t, carry, i4v=i4v):
            rowv, colv = carry
            m = plsc.load_gather(idx2_v, [rowv, colv]) * EMB
            base = t * NVEC
            i4v[pl.ds(base, NVEC)] = m
            i4v[pl.ds(GSEC + base, NVEC)] = m + 1
            i4v[pl.ds(2 * GSEC + base, NVEC)] = m + 2
            i4v[pl.ds(3 * GSEC + base, NVEC)] = m + 3
            colv = colv + NVEC
            wrap = colv >= HIST
            colv = jnp.where(wrap, colv - HIST, colv)
            return rowv + wrap.astype(jnp.int32), colv

        lax.fori_loop(0, GSEC // NVEC, exp_body, (zero + g * GB, lanes))
        copies.append(pltpu.async_copy(ab_table.at[i4v], rows_g[g], sem_g[g]))

    for v in range(O_PER_W // NVEC):
        wvals = plsc.load_gather(widx_v, [v * 4 + sub, zero])
        widx4_v[pl.ds(v * NVEC, NVEC)] = wvals * EMB + elem
    wcopy = pltpu.async_copy(wp_table.at[widx4_v], wrows_v, sem=wsem)
    wcopy.wait()

    for g in range(GROUPS):
        copies[g].wait()
        rv = rows_g[g]
        for v in range(GB * EMB // NVEC):
            o = g * GB * EMB + v * NVEC
            acc = wrows_v[pl.ds(o, NVEC)]
            ebase = elem * GSEC + (v * 4 + sub) * HIST

            def body(i, acc, rv=rv, ebase=ebase):
                i4 = i * 4
                g0 = plsc.load_gather(rv, [ebase + i4])
                g1 = plsc.load_gather(rv, [ebase + (i4 + 1)])
                g2 = plsc.load_gather(rv, [ebase + (i4 + 2)])
                g3 = plsc.load_gather(rv, [ebase + (i4 + 3)])
                return acc + ((g0 + g1) + (g2 + g3))

            acc = lax.fori_loop(0, HIST // 4, body, acc)
            out_v[pl.ds(o, NVEC)] = acc

    pltpu.sync_copy(out_v, x_out.at[pl.ds(wid * O_PER_W, O_PER_W)])


def _sc_pool(ab_idx, wp_idx, at_flat, wt_flat):
    mesh = plsc.VectorSubcoreMesh(core_axis_name="c", subcore_axis_name="s",
                                  num_cores=NUM_CORES,
                                  num_subcores=NUM_SUBCORES)
    fn = pl.kernel(
        _sc_pool_body,
        out_type=jax.ShapeDtypeStruct((B * EMB,), jnp.float32),
        mesh=mesh,
        compiler_params=pltpu.CompilerParams(needs_layout_passes=False),
        scratch_types=(
            [pltpu.VMEM((B_PER_W, HIST), jnp.int32)]
            + [pltpu.VMEM((E_PER_W // 4,), jnp.int32)] * 4
            + [pltpu.VMEM((E_PER_W // 4,), jnp.float32)] * 4
            + [
                pltpu.VMEM((B_PER_W, 1), jnp.int32),
                pltpu.VMEM((O_PER_W,), jnp.int32),
                pltpu.VMEM((O_PER_W,), jnp.float32),
                pltpu.VMEM((O_PER_W,), jnp.float32),
            ]
            + [pltpu.SemaphoreType.DMA] * 5
        ),
    )
    return fn(at_flat, ab_idx, wt_flat, wp_idx)


B_TILE = 16


def _tc_linear_body(x_ref, w_ref, b_ref, o_ref):
    o_ref[...] = lax.dot_general(
        x_ref[...], w_ref[...], (((1,), (0,)), ((), ())),
        preferred_element_type=jnp.float32) + b_ref[...]


def _tc_linear(x2d, W, b2d):
    nb = B // B_TILE
    return pl.pallas_call(
        _tc_linear_body,
        grid=(nb,),
        in_specs=[
            pl.BlockSpec((B_TILE, EMB), lambda i: (i, 0)),
            pl.BlockSpec((EMB, VOCAB), lambda i: (0, 0)),
            pl.BlockSpec((1, VOCAB), lambda i: (0, 0)),
        ],
        out_specs=pl.BlockSpec((B_TILE, VOCAB), lambda i: (i, 0)),
        out_shape=jax.ShapeDtypeStruct((B, VOCAB), jnp.float32),
    )(x2d, W, b2d)


def kernel(abilities, weapons, ability_table, weapon_table, W, b):
    ab_idx = abilities if abilities.dtype == jnp.int32 else (
        abilities.astype(jnp.int32))
    wp_idx = weapons if weapons.dtype == jnp.int32 else (
        weapons.astype(jnp.int32))
    x = _sc_pool(ab_idx, wp_idx,
                 ability_table.reshape(-1), weapon_table.reshape(-1))
    x2d = x.reshape(B, EMB)
    return _tc_linear(x2d, W, b.reshape(1, VOCAB))
